# baseline (speedup 1.0000x reference)
import jax
import jax.numpy as jnp
from jax.experimental import pallas as pl
from jax.experimental.pallas import tpu as pltpu

NKPT = 17
BS, T = 32, 16
CIN = 256
PIX = 64
FEAT = 256
HID = 64
ROWS = BS * NKPT
TPB = 2


def _gru_body(x_ref, x1_ref, cw2_ref, wxc_ref, bxc_ref, wgh_ref, wch_ref,
              cd_ref, cup_ref, cdn_ref, hs_ref, h_ref):
    g = pl.program_id(0)

    @pl.when(g == 0)
    def _init():
        h_ref[...] = jnp.zeros_like(h_ref)

    cw2 = cw2_ref[...]
    wxc = wxc_ref[...]
    bxc = bxc_ref[...]
    wgh = wgh_ref[...]
    wch = wch_ref[...]
    cd = cd_ref[...]
    cup = cup_ref[...]
    cdn = cdn_ref[...]
    zrow = jnp.zeros((1, HID), jnp.float32)

    def a_stencil(v):
        up = jnp.concatenate([v[1:], zrow], axis=0)
        dn = jnp.concatenate([zrow, v[:-1]], axis=0)
        return cd * v + cup * up + cdn * dn

    h = h_ref[...]
    for i in range(TPB):
        blocks = []
        for b in range(BS):
            xr = x_ref if b < BS // 2 else x1_ref
            fb = jnp.dot(cw2, xr[i, b % (BS // 2)],
                         preferred_element_type=jnp.float32)
            nb = jnp.concatenate(
                [fb[0:17], fb[17:34], fb[34:51], fb[51:68]], axis=1)
            blocks.append(nb)
        nodes = jnp.concatenate(blocks, axis=0)

        pgc = jnp.dot(nodes, wxc,
                      preferred_element_type=jnp.float32) + bxc
        pg = pgc[:, :2 * HID]
        pcx = pgc[:, 2 * HID:]

        gates = jax.nn.sigmoid(
            pg + jnp.dot(a_stencil(h), wgh,
                         preferred_element_type=jnp.float32))
        r = gates[:, :HID]
        u = gates[:, HID:]
        c = jnp.tanh(
            pcx + jnp.dot(a_stencil(r * h), wch,
                          preferred_element_type=jnp.float32))
        h = u * h + (1.0 - u) * c
        hs_ref[i] = h
    h_ref[...] = h


def _fc_body(a_ref, fc1w_ref, fc1b_ref, fc2w_ref, fc2b_ref, out_ref):
    f1 = jnp.dot(a_ref[...], fc1w_ref[...],
                 preferred_element_type=jnp.float32) + fc1b_ref[...]
    out_ref[...] = jnp.dot(f1, fc2w_ref[...],
                           preferred_element_type=jnp.float32) + fc2b_ref[...]


def kernel(x, conv_w, conv_b, Wg, bg, Wc, bc, fc1_w, fc1_b, fc2_w, fc2_b, A_hat):
    f32 = jnp.float32
    x2 = jnp.transpose(x.reshape(BS, T, CIN, PIX), (1, 0, 2, 3))

    M68 = jnp.kron(A_hat, jnp.eye(4, dtype=f32))
    cw2 = (M68 @ conv_w).reshape(NKPT, 4, CIN)
    cw2 = cw2.swapaxes(0, 1).reshape(4 * NKPT, CIN)
    cb2 = M68 @ conv_b

    Wxc = jnp.concatenate([Wg[:FEAT], Wc[:FEAT]], axis=1)
    Wgh = Wg[FEAT:]
    Wch = Wc[FEAT:]

    pb = cb2.reshape(NKPT, 4) @ Wxc.reshape(4, PIX, 192).sum(axis=1)
    bxc = jnp.tile(pb, (BS, 1)) + jnp.concatenate([bg, bc])[None, :]

    d0 = jnp.diagonal(A_hat)
    dup = jnp.concatenate([jnp.diagonal(A_hat, 1), jnp.zeros((1,), f32)])
    ddn = jnp.concatenate([jnp.zeros((1,), f32), jnp.diagonal(A_hat, -1)])
    cd = jnp.tile(d0, BS)[:, None]
    cup = jnp.tile(dup, BS)[:, None]
    cdn = jnp.tile(ddn, BS)[:, None]

    full = lambda shape: pl.BlockSpec(shape, lambda g: tuple(0 for _ in shape))

    hs = pl.pallas_call(
        _gru_body,
        grid=(T // TPB,),
        in_specs=[
            pl.BlockSpec((TPB, BS // 2, CIN, PIX), lambda g: (g, 0, 0, 0)),
            pl.BlockSpec((TPB, BS // 2, CIN, PIX), lambda g: (g, 1, 0, 0)),
            full((4 * NKPT, CIN)),
            full((CIN, 192)),
            full((ROWS, 192)),
            full((HID, 2 * HID)),
            full((HID, HID)),
            full((ROWS, 1)),
            full((ROWS, 1)),
            full((ROWS, 1)),
        ],
        out_specs=pl.BlockSpec((TPB, ROWS, HID), lambda g: (g, 0, 0)),
        out_shape=jax.ShapeDtypeStruct((T, ROWS, HID), f32),
        scratch_shapes=[pltpu.VMEM((ROWS, HID), f32)],
    )(x2, x2, cw2, Wxc, bxc, Wgh, Wch, cd, cup, cdn)

    fc_in = hs.reshape(T * BS, NKPT * HID)

    out = pl.pallas_call(
        _fc_body,
        out_shape=jax.ShapeDtypeStruct((T * BS, NKPT * 2), f32),
    )(fc_in, fc1_w, fc1_b[None, :], fc2_w, fc2_b[None, :])

    return out.reshape(T, BS, NKPT * 2).swapaxes(0, 1)


# speedup vs baseline: 1.0275x; 1.0275x over previous
import jax
import jax.numpy as jnp
from jax.experimental import pallas as pl
from jax.experimental.pallas import tpu as pltpu

NKPT = 17
BS, T = 32, 16
CIN = 256
PIX = 64
FEAT = 256
HID = 64
ROWS = BS * NKPT
TPB = 2


def _gru_body(x_ref, x1_ref, cw2_ref, wxc_ref, bxc_ref, wgh_ref, wch_ref,
              cd_ref, cup_ref, cdn_ref, hs_ref, h_ref):
    g = pl.program_id(0)

    @pl.when(g == 0)
    def _init():
        h_ref[...] = jnp.zeros_like(h_ref)

    cw2 = cw2_ref[...]
    wxc = wxc_ref[...]
    bxc = bxc_ref[...]
    wgh = wgh_ref[...]
    wch = wch_ref[...]
    cd = cd_ref[...]
    cup = cup_ref[...]
    cdn = cdn_ref[...]
    zrow = jnp.zeros((1, HID), jnp.float32)

    def a_stencil(v):
        up = jnp.concatenate([v[1:], zrow], axis=0)
        dn = jnp.concatenate([zrow, v[:-1]], axis=0)
        return cd * v + cup * up + cdn * dn

    h = h_ref[...]
    for i in range(TPB):
        blocks = []
        for b in range(BS):
            xr = x_ref if b < BS // 2 else x1_ref
            fb = jnp.dot(cw2, xr[i, b % (BS // 2)],
                         preferred_element_type=jnp.float32)
            nb = jnp.concatenate(
                [fb[0:17], fb[17:34], fb[34:51], fb[51:68]], axis=1)
            blocks.append(nb)
        nodes = jnp.concatenate(blocks, axis=0)

        pgc = jnp.dot(nodes, wxc,
                      preferred_element_type=jnp.float32) + bxc
        pg = pgc[:, :2 * HID]
        pcx = pgc[:, 2 * HID:]

        gates = jax.nn.sigmoid(
            pg + jnp.dot(a_stencil(h), wgh,
                         preferred_element_type=jnp.float32))
        r = gates[:, :HID]
        u = gates[:, HID:]
        c = jnp.tanh(
            pcx + jnp.dot(a_stencil(r * h), wch,
                          preferred_element_type=jnp.float32))
        h = u * h + (1.0 - u) * c
        hs_ref[i] = h
    h_ref[...] = h


def _fc_body(a_ref, fc1w_ref, fc1b_ref, fc2w_ref, fc2b_ref, out_ref):
    f1 = jnp.dot(a_ref[...], fc1w_ref[...],
                 preferred_element_type=jnp.float32) + fc1b_ref[...]
    out_ref[...] = jnp.dot(f1, fc2w_ref[...],
                           preferred_element_type=jnp.float32) + fc2b_ref[...]


def kernel(x, conv_w, conv_b, Wg, bg, Wc, bc, fc1_w, fc1_b, fc2_w, fc2_b, A_hat):
    f32 = jnp.float32
    x2 = jnp.transpose(x.reshape(BS, T, CIN, PIX), (1, 0, 2, 3)).astype(jnp.bfloat16)

    M68 = jnp.kron(A_hat, jnp.eye(4, dtype=f32))
    cw2 = (M68 @ conv_w).reshape(NKPT, 4, CIN)
    cw2 = cw2.swapaxes(0, 1).reshape(4 * NKPT, CIN).astype(jnp.bfloat16)
    cb2 = M68 @ conv_b

    Wxc = jnp.concatenate([Wg[:FEAT], Wc[:FEAT]], axis=1)
    Wgh = Wg[FEAT:]
    Wch = Wc[FEAT:]

    pb = cb2.reshape(NKPT, 4) @ Wxc.reshape(4, PIX, 192).sum(axis=1)
    bxc = jnp.tile(pb, (BS, 1)) + jnp.concatenate([bg, bc])[None, :]

    d0 = jnp.diagonal(A_hat)
    dup = jnp.concatenate([jnp.diagonal(A_hat, 1), jnp.zeros((1,), f32)])
    ddn = jnp.concatenate([jnp.zeros((1,), f32), jnp.diagonal(A_hat, -1)])
    cd = jnp.tile(d0, BS)[:, None]
    cup = jnp.tile(dup, BS)[:, None]
    cdn = jnp.tile(ddn, BS)[:, None]

    full = lambda shape: pl.BlockSpec(shape, lambda g: tuple(0 for _ in shape))

    hs = pl.pallas_call(
        _gru_body,
        grid=(T // TPB,),
        in_specs=[
            pl.BlockSpec((TPB, BS // 2, CIN, PIX), lambda g: (g, 0, 0, 0)),
            pl.BlockSpec((TPB, BS // 2, CIN, PIX), lambda g: (g, 1, 0, 0)),
            full((4 * NKPT, CIN)),
            full((CIN, 192)),
            full((ROWS, 192)),
            full((HID, 2 * HID)),
            full((HID, HID)),
            full((ROWS, 1)),
            full((ROWS, 1)),
            full((ROWS, 1)),
        ],
        out_specs=pl.BlockSpec((TPB, ROWS, HID), lambda g: (g, 0, 0)),
        out_shape=jax.ShapeDtypeStruct((T, ROWS, HID), f32),
        scratch_shapes=[pltpu.VMEM((ROWS, HID), f32)],
    )(x2, x2, cw2, Wxc, bxc, Wgh, Wch, cd, cup, cdn)

    fc_in = hs.reshape(T * BS, NKPT * HID)

    out = pl.pallas_call(
        _fc_body,
        out_shape=jax.ShapeDtypeStruct((T * BS, NKPT * 2), f32),
    )(fc_in, fc1_w, fc1_b[None, :], fc2_w, fc2_b[None, :])

    return out.reshape(T, BS, NKPT * 2).swapaxes(0, 1)


# speedup vs baseline: 1.0279x; 1.0004x over previous
import jax
import jax.numpy as jnp
from jax.experimental import pallas as pl
from jax.experimental.pallas import tpu as pltpu

NKPT = 17
BS, T = 32, 16
CIN = 256
PIX = 64
FEAT = 256
HID = 64
ROWS = BS * NKPT
TPB = 2


def _gru_body(x_ref, x1_ref, cw2_ref, wxc_ref, bxc_ref, wgh_ref, wch_ref,
              cd_ref, cup_ref, cdn_ref, hs_ref, h_ref):
    g = pl.program_id(0)

    @pl.when(g == 0)
    def _init():
        h_ref[...] = jnp.zeros_like(h_ref)

    cw2 = cw2_ref[...]
    wxc = wxc_ref[...]
    bxc = bxc_ref[...]
    wgh = wgh_ref[...]
    wch = wch_ref[...]
    cd = cd_ref[...]
    cup = cup_ref[...]
    cdn = cdn_ref[...]
    zrow = jnp.zeros((1, HID), jnp.float32)

    def a_stencil(v):
        up = jnp.concatenate([v[1:], zrow], axis=0)
        dn = jnp.concatenate([zrow, v[:-1]], axis=0)
        return cd * v + cup * up + cdn * dn

    h = h_ref[...]
    for i in range(TPB):
        blocks = []
        for b in range(BS):
            xr = x_ref if b < BS // 2 else x1_ref
            fb = jnp.dot(cw2, xr[i, b % (BS // 2)],
                         preferred_element_type=jnp.float32)
            nb = jnp.concatenate(
                [fb[0:17], fb[17:34], fb[34:51], fb[51:68]], axis=1)
            blocks.append(nb)
        nodes = jnp.concatenate(blocks, axis=0)

        pgc = jnp.dot(nodes, wxc,
                      preferred_element_type=jnp.float32) + bxc
        pg = pgc[:, :2 * HID]
        pcx = pgc[:, 2 * HID:]

        gates = jax.nn.sigmoid(
            pg + jnp.dot(a_stencil(h), wgh,
                         preferred_element_type=jnp.float32))
        r = gates[:, :HID]
        u = gates[:, HID:]
        c = jnp.tanh(
            pcx + jnp.dot(a_stencil(r * h), wch,
                          preferred_element_type=jnp.float32))
        h = u * h + (1.0 - u) * c
        hs_ref[i] = h
    h_ref[...] = h


def _fc_body(a_ref, fc1w_ref, fc1b_ref, fc2w_ref, fc2b_ref, out_ref):
    f1 = jnp.dot(a_ref[...], fc1w_ref[...],
                 preferred_element_type=jnp.float32) + fc1b_ref[...]
    out_ref[...] = jnp.dot(f1, fc2w_ref[...],
                           preferred_element_type=jnp.float32) + fc2b_ref[...]


def kernel(x, conv_w, conv_b, Wg, bg, Wc, bc, fc1_w, fc1_b, fc2_w, fc2_b, A_hat):
    f32 = jnp.float32
    xb16 = x.astype(jnp.bfloat16)
    x2 = jnp.transpose(xb16.reshape(BS, T, CIN, PIX), (1, 0, 2, 3))

    M68 = jnp.kron(A_hat, jnp.eye(4, dtype=f32))
    cw2 = (M68 @ conv_w).reshape(NKPT, 4, CIN)
    cw2 = cw2.swapaxes(0, 1).reshape(4 * NKPT, CIN).astype(jnp.bfloat16)
    cb2 = M68 @ conv_b

    Wxc = jnp.concatenate([Wg[:FEAT], Wc[:FEAT]], axis=1)
    Wgh = Wg[FEAT:]
    Wch = Wc[FEAT:]

    pb = cb2.reshape(NKPT, 4) @ Wxc.reshape(4, PIX, 192).sum(axis=1)
    bxc = jnp.tile(pb, (BS, 1)) + jnp.concatenate([bg, bc])[None, :]

    d0 = jnp.diagonal(A_hat)
    dup = jnp.concatenate([jnp.diagonal(A_hat, 1), jnp.zeros((1,), f32)])
    ddn = jnp.concatenate([jnp.zeros((1,), f32), jnp.diagonal(A_hat, -1)])
    cd = jnp.tile(d0, BS)[:, None]
    cup = jnp.tile(dup, BS)[:, None]
    cdn = jnp.tile(ddn, BS)[:, None]

    full = lambda shape: pl.BlockSpec(shape, lambda g: tuple(0 for _ in shape))

    hs = pl.pallas_call(
        _gru_body,
        grid=(T // TPB,),
        in_specs=[
            pl.BlockSpec((TPB, BS // 2, CIN, PIX), lambda g: (g, 0, 0, 0)),
            pl.BlockSpec((TPB, BS // 2, CIN, PIX), lambda g: (g, 1, 0, 0)),
            full((4 * NKPT, CIN)),
            full((CIN, 192)),
            full((ROWS, 192)),
            full((HID, 2 * HID)),
            full((HID, HID)),
            full((ROWS, 1)),
            full((ROWS, 1)),
            full((ROWS, 1)),
        ],
        out_specs=pl.BlockSpec((TPB, ROWS, HID), lambda g: (g, 0, 0)),
        out_shape=jax.ShapeDtypeStruct((T, ROWS, HID), f32),
        scratch_shapes=[pltpu.VMEM((ROWS, HID), f32)],
    )(x2, x2, cw2, Wxc, bxc, Wgh, Wch, cd, cup, cdn)

    fc_in = hs.reshape(T * BS, NKPT * HID)

    out = pl.pallas_call(
        _fc_body,
        out_shape=jax.ShapeDtypeStruct((T * BS, NKPT * 2), f32),
    )(fc_in, fc1_w, fc1_b[None, :], fc2_w, fc2_b[None, :])

    return out.reshape(T, BS, NKPT * 2).swapaxes(0, 1)
